# baseline (speedup 1.0000x reference)
import jax
import jax.numpy as jnp
from jax import lax
from jax.experimental import pallas as pl
from jax.experimental.pallas import tpu as pltpu
from jax.experimental.pallas import tpu_sc as plsc

_N = 512
_P = 64
_L = 16
_G = 8
_NB = _N // _L
_QC = _N // (_L * _G)
_NW = 32
_QSC = 448


def _lane_gather(vec, sel):
    dnums = lax.GatherDimensionNumbers(
        offset_dims=(), collapsed_slice_dims=(0,), start_index_map=(0,))
    return lax.gather(vec, sel[:, None], dnums, (1,),
                      mode=lax.GatherScatterMode.PROMISE_IN_BOUNDS)


def _sl1v(a, b):
    d = a - b
    ad = jnp.abs(d)
    return jnp.where(ad < 1.0, 0.5 * d * d, ad - 0.5)


_PPT = 1
_OVERLAP_PROBE = False


def _sc_body(x_hbm, t_hbm, w_hbm, out_hbm, *refs):
    xtbufs = refs[:2 * _PPT]
    k2x, k2y, k2z, cc, wbuf, outv = refs[2 * _PPT:2 * _PPT + 6]
    sems = refs[2 * _PPT + 6:]
    c = lax.axis_index("c")
    s = lax.axis_index("s")
    wid = s * 2 + c
    copies = []
    for pi in range(_PPT):
        part = wid * _PPT + pi
        copies.append((
            pltpu.async_copy(x_hbm.at[part], xtbufs[2 * pi], sems[2 * pi]),
            pltpu.async_copy(t_hbm.at[part], xtbufs[2 * pi + 1],
                             sems[2 * pi + 1]),
        ))
    pltpu.sync_copy(w_hbm, wbuf)
    iota = lax.iota(jnp.int32, _L)

    def do_part(part, xall, tall, waiters, grand, lossc_acc):
        for wtr in waiters:
            wtr.wait()

        def setup(i, _):
            tx = tall[pl.ds(i * _L, _L)]
            ty = tall[pl.ds(_N + i * _L, _L)]
            tz = tall[pl.ds(2 * _N + i * _L, _L)]
            k2x[pl.ds(i * _L, _L)] = tx * (-2.0)
            k2y[pl.ds(i * _L, _L)] = ty * (-2.0)
            k2z[pl.ds(i * _L, _L)] = tz * (-2.0)
            cc[pl.ds(i * _L, _L)] = tx * tx + ty * ty + tz * tz
            return 0

        lax.fori_loop(0, _NB, setup, 0)

        def make_chunk(Gc, qoff):
            def chunk_body(qc, acc):
                qbase = qoff + qc * (_L * Gc)
                qxs = tuple(xall[pl.ds(qbase + g * _L, _L)]
                            for g in range(Gc))
                qys = tuple(xall[pl.ds(_N + qbase + g * _L, _L)]
                            for g in range(Gc))
                qzs = tuple(xall[pl.ds(2 * _N + qbase + g * _L, _L)]
                            for g in range(Gc))
                dmin0 = tuple(jnp.full((_L,), jnp.inf, jnp.float32)
                              for _ in range(Gc))
                imin0 = tuple(jnp.zeros((_L,), jnp.int32)
                              for _ in range(Gc))

                def key_body(kb, dc):
                    dmins, imins = dc
                    dmins = list(dmins)
                    imins = list(imins)
                    base = kb * _L
                    txv = k2x[pl.ds(base, _L)]
                    tyv = k2y[pl.ds(base, _L)]
                    tzv = k2z[pl.ds(base, _L)]
                    tcv = cc[pl.ds(base, _L)]
                    kbase = jnp.full((_L,), base, jnp.int32)
                    for j in range(_L):
                        sel = jnp.full((_L,), j, jnp.int32)
                        bx = _lane_gather(txv, sel)
                        by = _lane_gather(tyv, sel)
                        bz = _lane_gather(tzv, sel)
                        bc = _lane_gather(tcv, sel)
                        idxv = kbase + j
                        for g in range(Gc):
                            d = (qxs[g] * bx + qys[g] * by + qzs[g] * bz
                                 + bc)
                            m = d < dmins[g]
                            dmins[g] = jnp.minimum(d, dmins[g])
                            imins[g] = jnp.where(m, idxv, imins[g])
                    return (tuple(dmins), tuple(imins))

                _, imins = lax.fori_loop(0, _NB, key_body, (dmin0, imin0))

                for g in range(Gc):
                    im = imins[g]
                    gx = plsc.load_gather(tall, [im])
                    gy = plsc.load_gather(tall, [im + _N])
                    gz = plsc.load_gather(tall, [im + 2 * _N])
                    acc = (acc + _sl1v(qxs[g], gx) + _sl1v(qys[g], gy)
                           + _sl1v(qzs[g], gz))
                return acc
            return chunk_body

        nfull = _QSC // (_L * _G)
        acc = lax.fori_loop(0, nfull, make_chunk(_G, 0),
                            jnp.zeros((_L,), jnp.float32))
        rem = _QSC - nfull * _L * _G
        if rem:
            acc = make_chunk(rem // _L, nfull * _L * _G)(0, acc)
        wsp = plsc.load_gather(wbuf, [jnp.full((_L,), part, jnp.int32)])
        grand = grand + acc * wsp

        def cent(i, c6):
            sx, sy, sz, tx_, ty_, tz_ = c6
            sx = sx + xall[pl.ds(i * _L, _L)]
            sy = sy + xall[pl.ds(_N + i * _L, _L)]
            sz = sz + xall[pl.ds(2 * _N + i * _L, _L)]
            tx_ = tx_ + tall[pl.ds(i * _L, _L)]
            ty_ = ty_ + tall[pl.ds(_N + i * _L, _L)]
            tz_ = tz_ + tall[pl.ds(2 * _N + i * _L, _L)]
            return (sx, sy, sz, tx_, ty_, tz_)

        z = jnp.zeros((_L,), jnp.float32)
        sx, sy, sz, tcx, tcy, tcz = lax.fori_loop(0, _NB, cent,
                                                  (z, z, z, z, z, z))
        inv = 1.0 / _N
        dx = (jnp.sum(sx) - jnp.sum(tcx)) * inv
        dy = (jnp.sum(sy) - jnp.sum(tcy)) * inv
        dz = (jnp.sum(sz) - jnp.sum(tcz)) * inv
        cdiff = jnp.where(iota == 0, dx,
                          jnp.where(iota == 1, dy,
                                    jnp.where(iota == 2, dz, 0.0)))
        lossc_acc = lossc_acc + _sl1v(cdiff, jnp.zeros((_L,), jnp.float32))
        return grand, lossc_acc

    grand = jnp.zeros((_L,), jnp.float32)
    lossc_acc = jnp.zeros((_L,), jnp.float32)
    for pi in range(_PPT):
        grand, lossc_acc = do_part(wid * _PPT + pi, xtbufs[2 * pi],
                                   xtbufs[2 * pi + 1], copies[pi],
                                   grand, lossc_acc)
    loss_p = jnp.sum(grand) * (1.0 / (_N * 3.0 * 4.0))
    lossc_p = jnp.sum(lossc_acc) * (1.0 / 12.0)
    outv[...] = jnp.where(iota == 0, loss_p,
                          jnp.where(iota == 1, lossc_p, 0.0))
    pltpu.sync_copy(outv, out_hbm.at[wid])


_TPB = 2


def _tc_nn_sl1_sum(xT, tT):
    N = xT.shape[1]
    M = tT.shape[1]
    G = jax.lax.dot_general(xT, tT, (((0,), (0,)), ((), ())),
                            preferred_element_type=jnp.float32)
    c = jnp.sum(tT * tT, axis=0)
    D = c[None, :] - (G + G)
    minD = jnp.min(D, axis=1)
    iota_f = jax.lax.broadcasted_iota(jnp.int32, (N, M), 1
                                      ).astype(jnp.float32)
    matches = D <= minD[:, None]
    idx_f = jnp.min(jnp.where(matches, iota_f, jnp.float32(M)), axis=1)
    ohT = (jax.lax.broadcasted_iota(jnp.int32, (M, N), 0
                                    ).astype(jnp.float32)
           == idx_f[None, :]).astype(jnp.float32)
    tagpT = jax.lax.dot_general(tT, ohT, (((1,), (0,)), ((), ())),
                                preferred_element_type=jnp.float32)
    return jnp.sum(_sl1v(xT, tagpT))


def _tc_body(xT_ref, tT_ref, w_ref, xq2_ref, tT2_ref, w2_ref,
             loss_ref, lossc_ref):
    i = pl.program_id(0)
    part_loss = jnp.float32(0.0)
    part_lossc = jnp.float32(0.0)
    for j in range(_TPB):
        xT = xT_ref[j]
        tT = tT_ref[j]
        N = xT.shape[1]
        M = tT.shape[1]
        w = w_ref[j, 0, 0]
        part_loss += _tc_nn_sl1_sum(xT, tT) / (_N * 3.0) * w / 4.0
        sx = jnp.sum(xT, axis=1) / N
        st = jnp.sum(tT, axis=1) / M
        part_lossc += jnp.sum(_sl1v(sx, st)) / 12.0
        if _QSC < _N:
            w2 = w2_ref[j, 0, 0]
            part_loss += (_tc_nn_sl1_sum(xq2_ref[j], tT2_ref[j])
                          / (_N * 3.0) * w2 / 4.0)

    @pl.when(i == 0)
    def _():
        loss_ref[...] = jnp.zeros((1, 1), jnp.float32)
        lossc_ref[...] = jnp.zeros((1, 1), jnp.float32)

    loss_ref[...] = loss_ref[...] + part_loss
    lossc_ref[...] = lossc_ref[...] + part_lossc


def kernel(X_v, target_X_v, weights):
    B, K, N, D = X_v.shape
    P = B * K
    xT3 = jnp.transpose(X_v, (0, 1, 3, 2)).reshape(P, D, N)
    tT3 = jnp.transpose(target_X_v, (0, 1, 3, 2)).reshape(P, D, N)
    w = weights.reshape(P)
    S = _NW * _PPT
    xT = xT3.reshape(P, D * N)
    tT = tT3.reshape(P, D * N)
    mesh = plsc.VectorSubcoreMesh(core_axis_name="c", subcore_axis_name="s")
    scratch = ([pltpu.VMEM((D * N,), jnp.float32)] * (2 * _PPT)
               + [pltpu.VMEM((N,), jnp.float32)] * 4
               + [pltpu.VMEM((S,), jnp.float32),
                  pltpu.VMEM((_L,), jnp.float32)]
               + [pltpu.SemaphoreType.DMA] * (2 * _PPT))
    sc_call = pl.kernel(
        _sc_body,
        out_type=jax.ShapeDtypeStruct((_NW, _L), jnp.float32),
        mesh=mesh,
        scratch_types=scratch,
        compiler_params=pltpu.CompilerParams(needs_layout_passes=False),
    )
    TS = 0 if _OVERLAP_PROBE else S
    if TS < P:
        R = P - TS
        QR = N - _QSC
        xq2 = xT3[:S, :, _QSC:]
        tT2 = tT3[:S]
        w2 = w[:S].reshape(S, 1, 1)
        lr, lcr = pl.pallas_call(
            _tc_body,
            grid=(R // _TPB,),
            in_specs=[
                pl.BlockSpec((_TPB, D, N), lambda i: (i, 0, 0)),
                pl.BlockSpec((_TPB, D, N), lambda i: (i, 0, 0)),
                pl.BlockSpec((_TPB, 1, 1), lambda i: (i, 0, 0)),
                pl.BlockSpec((_TPB, D, QR), lambda i: (i, 0, 0)),
                pl.BlockSpec((_TPB, D, N), lambda i: (i, 0, 0)),
                pl.BlockSpec((_TPB, 1, 1), lambda i: (i, 0, 0)),
            ],
            out_specs=[
                pl.BlockSpec((1, 1), lambda i: (0, 0)),
                pl.BlockSpec((1, 1), lambda i: (0, 0)),
            ],
            out_shape=[
                jax.ShapeDtypeStruct((1, 1), jnp.float32),
                jax.ShapeDtypeStruct((1, 1), jnp.float32),
            ],
            compiler_params=pltpu.CompilerParams(
                dimension_semantics=("arbitrary",),
            ),
        )(xT3[TS:], tT3[TS:], w[TS:].reshape(R, 1, 1), xq2, tT2, w2)
    else:
        lr = lcr = jnp.zeros((1, 1), jnp.float32)
    partials = sc_call(xT[:S], tT[:S], w[:S])
    if _OVERLAP_PROBE:
        loss = 0.5 * (jnp.sum(partials[:, 0]) + lr[0, 0])
        lossc = 0.5 * (jnp.sum(partials[:, 1]) + lcr[0, 0])
    else:
        loss = jnp.sum(partials[:, 0]) + lr[0, 0]
        lossc = jnp.sum(partials[:, 1]) + lcr[0, 0]
    return loss, lossc


# speedup vs baseline: 1.0325x; 1.0325x over previous
import jax
import jax.numpy as jnp
from jax import lax
from jax.experimental import pallas as pl
from jax.experimental.pallas import tpu as pltpu
from jax.experimental.pallas import tpu_sc as plsc

_N = 512
_P = 64
_L = 16
_G = 8
_NB = _N // _L
_QC = _N // (_L * _G)
_NW = 32
_QSC = 512


def _lane_gather(vec, sel):
    dnums = lax.GatherDimensionNumbers(
        offset_dims=(), collapsed_slice_dims=(0,), start_index_map=(0,))
    return lax.gather(vec, sel[:, None], dnums, (1,),
                      mode=lax.GatherScatterMode.PROMISE_IN_BOUNDS)


def _sl1v(a, b):
    d = a - b
    ad = jnp.abs(d)
    return jnp.where(ad < 1.0, 0.5 * d * d, ad - 0.5)


_PPT = 1
_OVERLAP_PROBE = False


def _sc_body(x_hbm, t_hbm, w_hbm, out_hbm, *refs):
    xtbufs = refs[:2 * _PPT]
    k2x, k2y, k2z, cc, wbuf, outv = refs[2 * _PPT:2 * _PPT + 6]
    sems = refs[2 * _PPT + 6:]
    c = lax.axis_index("c")
    s = lax.axis_index("s")
    wid = s * 2 + c
    copies = []
    for pi in range(_PPT):
        part = wid * _PPT + pi
        copies.append((
            pltpu.async_copy(x_hbm.at[part], xtbufs[2 * pi], sems[2 * pi]),
            pltpu.async_copy(t_hbm.at[part], xtbufs[2 * pi + 1],
                             sems[2 * pi + 1]),
        ))
    pltpu.sync_copy(w_hbm, wbuf)
    iota = lax.iota(jnp.int32, _L)

    def do_part(part, xall, tall, waiters, grand, lossc_acc):
        for wtr in waiters:
            wtr.wait()

        def setup(i, _):
            tx = tall[pl.ds(i * _L, _L)]
            ty = tall[pl.ds(_N + i * _L, _L)]
            tz = tall[pl.ds(2 * _N + i * _L, _L)]
            k2x[pl.ds(i * _L, _L)] = tx * (-2.0)
            k2y[pl.ds(i * _L, _L)] = ty * (-2.0)
            k2z[pl.ds(i * _L, _L)] = tz * (-2.0)
            cc[pl.ds(i * _L, _L)] = tx * tx + ty * ty + tz * tz
            return 0

        lax.fori_loop(0, _NB, setup, 0)

        def make_chunk(Gc, qoff):
            def chunk_body(qc, acc):
                qbase = qoff + qc * (_L * Gc)
                qxs = tuple(xall[pl.ds(qbase + g * _L, _L)]
                            for g in range(Gc))
                qys = tuple(xall[pl.ds(_N + qbase + g * _L, _L)]
                            for g in range(Gc))
                qzs = tuple(xall[pl.ds(2 * _N + qbase + g * _L, _L)]
                            for g in range(Gc))
                dmin0 = tuple(jnp.full((_L,), jnp.inf, jnp.float32)
                              for _ in range(Gc))
                imin0 = tuple(jnp.zeros((_L,), jnp.int32)
                              for _ in range(Gc))

                def key_body(kb, dc):
                    dmins, imins = dc
                    dmins = list(dmins)
                    imins = list(imins)
                    base = kb * _L
                    txv = k2x[pl.ds(base, _L)]
                    tyv = k2y[pl.ds(base, _L)]
                    tzv = k2z[pl.ds(base, _L)]
                    tcv = cc[pl.ds(base, _L)]
                    kbase = jnp.full((_L,), base, jnp.int32)
                    for j in range(_L):
                        sel = jnp.full((_L,), j, jnp.int32)
                        bx = _lane_gather(txv, sel)
                        by = _lane_gather(tyv, sel)
                        bz = _lane_gather(tzv, sel)
                        bc = _lane_gather(tcv, sel)
                        idxv = kbase + j
                        for g in range(Gc):
                            d = (qxs[g] * bx + qys[g] * by + qzs[g] * bz
                                 + bc)
                            m = d < dmins[g]
                            dmins[g] = jnp.minimum(d, dmins[g])
                            imins[g] = jnp.where(m, idxv, imins[g])
                    return (tuple(dmins), tuple(imins))

                _, imins = lax.fori_loop(0, _NB, key_body, (dmin0, imin0))

                for g in range(Gc):
                    im = imins[g]
                    gx = plsc.load_gather(tall, [im])
                    gy = plsc.load_gather(tall, [im + _N])
                    gz = plsc.load_gather(tall, [im + 2 * _N])
                    acc = (acc + _sl1v(qxs[g], gx) + _sl1v(qys[g], gy)
                           + _sl1v(qzs[g], gz))
                return acc
            return chunk_body

        nfull = _QSC // (_L * _G)
        acc = lax.fori_loop(0, nfull, make_chunk(_G, 0),
                            jnp.zeros((_L,), jnp.float32))
        rem = _QSC - nfull * _L * _G
        if rem:
            acc = make_chunk(rem // _L, nfull * _L * _G)(0, acc)
        wsp = plsc.load_gather(wbuf, [jnp.full((_L,), part, jnp.int32)])
        grand = grand + acc * wsp

        def cent(i, c6):
            sx, sy, sz, tx_, ty_, tz_ = c6
            sx = sx + xall[pl.ds(i * _L, _L)]
            sy = sy + xall[pl.ds(_N + i * _L, _L)]
            sz = sz + xall[pl.ds(2 * _N + i * _L, _L)]
            tx_ = tx_ + tall[pl.ds(i * _L, _L)]
            ty_ = ty_ + tall[pl.ds(_N + i * _L, _L)]
            tz_ = tz_ + tall[pl.ds(2 * _N + i * _L, _L)]
            return (sx, sy, sz, tx_, ty_, tz_)

        z = jnp.zeros((_L,), jnp.float32)
        sx, sy, sz, tcx, tcy, tcz = lax.fori_loop(0, _NB, cent,
                                                  (z, z, z, z, z, z))
        inv = 1.0 / _N
        dx = (jnp.sum(sx) - jnp.sum(tcx)) * inv
        dy = (jnp.sum(sy) - jnp.sum(tcy)) * inv
        dz = (jnp.sum(sz) - jnp.sum(tcz)) * inv
        cdiff = jnp.where(iota == 0, dx,
                          jnp.where(iota == 1, dy,
                                    jnp.where(iota == 2, dz, 0.0)))
        lossc_acc = lossc_acc + _sl1v(cdiff, jnp.zeros((_L,), jnp.float32))
        return grand, lossc_acc

    grand = jnp.zeros((_L,), jnp.float32)
    lossc_acc = jnp.zeros((_L,), jnp.float32)
    for pi in range(_PPT):
        grand, lossc_acc = do_part(wid * _PPT + pi, xtbufs[2 * pi],
                                   xtbufs[2 * pi + 1], copies[pi],
                                   grand, lossc_acc)
    loss_p = jnp.sum(grand) * (1.0 / (_N * 3.0 * 4.0))
    lossc_p = jnp.sum(lossc_acc) * (1.0 / 12.0)
    outv[...] = jnp.where(iota == 0, loss_p,
                          jnp.where(iota == 1, lossc_p, 0.0))
    pltpu.sync_copy(outv, out_hbm.at[wid])


_TPB = 2


def _tc_nn_sl1_sum(xT, tT):
    N = xT.shape[1]
    M = tT.shape[1]
    G = jax.lax.dot_general(xT, tT, (((0,), (0,)), ((), ())),
                            preferred_element_type=jnp.float32)
    c = jnp.sum(tT * tT, axis=0)
    D = c[None, :] - (G + G)
    minD = jnp.min(D, axis=1)
    iota_f = jax.lax.broadcasted_iota(jnp.int32, (N, M), 1
                                      ).astype(jnp.float32)
    matches = D <= minD[:, None]
    idx_f = jnp.min(jnp.where(matches, iota_f, jnp.float32(M)), axis=1)
    ohT = (jax.lax.broadcasted_iota(jnp.int32, (M, N), 0
                                    ).astype(jnp.float32)
           == idx_f[None, :]).astype(jnp.float32)
    tagpT = jax.lax.dot_general(tT, ohT, (((1,), (0,)), ((), ())),
                                preferred_element_type=jnp.float32)
    return jnp.sum(_sl1v(xT, tagpT))


def _tc_body(*refs):
    if _QSC < _N:
        (xT_ref, tT_ref, w_ref, xq2_ref, tT2_ref, w2_ref,
         loss_ref, lossc_ref) = refs
    else:
        xT_ref, tT_ref, w_ref, loss_ref, lossc_ref = refs
    i = pl.program_id(0)
    part_loss = jnp.float32(0.0)
    part_lossc = jnp.float32(0.0)
    for j in range(_TPB):
        xT = xT_ref[j]
        tT = tT_ref[j]
        N = xT.shape[1]
        M = tT.shape[1]
        w = w_ref[j, 0, 0]
        part_loss += _tc_nn_sl1_sum(xT, tT) / (_N * 3.0) * w / 4.0
        sx = jnp.sum(xT, axis=1) / N
        st = jnp.sum(tT, axis=1) / M
        part_lossc += jnp.sum(_sl1v(sx, st)) / 12.0
        if _QSC < _N:
            w2 = w2_ref[j, 0, 0]
            part_loss += (_tc_nn_sl1_sum(xq2_ref[j], tT2_ref[j])
                          / (_N * 3.0) * w2 / 4.0)

    @pl.when(i == 0)
    def _():
        loss_ref[...] = jnp.zeros((1, 1), jnp.float32)
        lossc_ref[...] = jnp.zeros((1, 1), jnp.float32)

    loss_ref[...] = loss_ref[...] + part_loss
    lossc_ref[...] = lossc_ref[...] + part_lossc


def kernel(X_v, target_X_v, weights):
    B, K, N, D = X_v.shape
    P = B * K
    xT3 = jnp.transpose(X_v, (0, 1, 3, 2)).reshape(P, D, N)
    tT3 = jnp.transpose(target_X_v, (0, 1, 3, 2)).reshape(P, D, N)
    w = weights.reshape(P)
    S = _NW * _PPT
    xT = xT3.reshape(P, D * N)
    tT = tT3.reshape(P, D * N)
    mesh = plsc.VectorSubcoreMesh(core_axis_name="c", subcore_axis_name="s")
    scratch = ([pltpu.VMEM((D * N,), jnp.float32)] * (2 * _PPT)
               + [pltpu.VMEM((N,), jnp.float32)] * 4
               + [pltpu.VMEM((S,), jnp.float32),
                  pltpu.VMEM((_L,), jnp.float32)]
               + [pltpu.SemaphoreType.DMA] * (2 * _PPT))
    sc_call = pl.kernel(
        _sc_body,
        out_type=jax.ShapeDtypeStruct((_NW, _L), jnp.float32),
        mesh=mesh,
        scratch_types=scratch,
        compiler_params=pltpu.CompilerParams(needs_layout_passes=False),
    )
    TS = 0 if _OVERLAP_PROBE else S
    if TS < P:
        R = P - TS
        in_specs = [
            pl.BlockSpec((_TPB, D, N), lambda i: (i, 0, 0)),
            pl.BlockSpec((_TPB, D, N), lambda i: (i, 0, 0)),
            pl.BlockSpec((_TPB, 1, 1), lambda i: (i, 0, 0)),
        ]
        tc_inputs = [xT3[TS:], tT3[TS:], w[TS:].reshape(R, 1, 1)]
        if _QSC < N:
            QR = N - _QSC
            in_specs += [
                pl.BlockSpec((_TPB, D, QR), lambda i: (i, 0, 0)),
                pl.BlockSpec((_TPB, D, N), lambda i: (i, 0, 0)),
                pl.BlockSpec((_TPB, 1, 1), lambda i: (i, 0, 0)),
            ]
            tc_inputs += [xT3[:S, :, _QSC:], tT3[:S],
                          w[:S].reshape(S, 1, 1)]
        lr, lcr = pl.pallas_call(
            _tc_body,
            grid=(R // _TPB,),
            in_specs=in_specs,
            out_specs=[
                pl.BlockSpec((1, 1), lambda i: (0, 0)),
                pl.BlockSpec((1, 1), lambda i: (0, 0)),
            ],
            out_shape=[
                jax.ShapeDtypeStruct((1, 1), jnp.float32),
                jax.ShapeDtypeStruct((1, 1), jnp.float32),
            ],
            compiler_params=pltpu.CompilerParams(
                dimension_semantics=("arbitrary",),
            ),
        )(*tc_inputs)
    else:
        lr = lcr = jnp.zeros((1, 1), jnp.float32)
    partials = sc_call(xT[:S], tT[:S], w[:S])
    if _OVERLAP_PROBE:
        loss = 0.5 * (jnp.sum(partials[:, 0]) + lr[0, 0])
        lossc = 0.5 * (jnp.sum(partials[:, 1]) + lcr[0, 0])
    else:
        loss = jnp.sum(partials[:, 0]) + lr[0, 0]
        lossc = jnp.sum(partials[:, 1]) + lcr[0, 0]
    return loss, lossc
